# baseline (speedup 1.0000x reference)
import functools

import jax
import jax.numpy as jnp
from jax import lax
from jax.experimental import pallas as pl
from jax.experimental.pallas import tpu as pltpu
from jax.experimental.pallas import tpu_sc as plsc

N = 10000
E = 320000
NUM_GRAPHS = 64
HIDDEN = 128
OUT = 11
PE = 8
IN = 128
LAYERS = 6
EPS = 1e-5

NC = 2
NS = 16
CHUNK = 128
NCH = 78
NEXTRA = 2
SRCBLK = 26
IO_TILES = 10
ROWS_PER_IO = N // IO_TILES

_MESH = plsc.VectorSubcoreMesh(core_axis_name="c", subcore_axis_name="s")
_HIGHEST = lax.Precision.HIGHEST


@functools.partial(
    pl.kernel,
    out_type=jax.ShapeDtypeStruct((NC, N, HIDDEN), jnp.float32),
    mesh=_MESH,
    scratch_types=[
        pltpu.VMEM((NCH, CHUNK), jnp.int32),
        pltpu.VMEM((1, CHUNK), jnp.int32),
        pltpu.VMEM((CHUNK, HIDDEN), jnp.float32),
        pltpu.VMEM_SHARED((N, HIDDEN), jnp.float32),
    ],
)
def _sc_degree(dst_hbm, edst_hbm, ones_hbm, zeros_hbm, out_hbm,
               dst_v, edst_v, ones_v, acc_sh):
    c = lax.axis_index("c")
    s = lax.axis_index("s")
    pltpu.sync_copy(dst_hbm.at[c, s], dst_v)
    pltpu.sync_copy(ones_hbm, ones_v)

    @pl.when(s < IO_TILES)
    def _():
        pltpu.sync_copy(zeros_hbm,
                        acc_sh.at[pl.ds(s * ROWS_PER_IO, ROWS_PER_IO)])

    plsc.subcore_barrier()

    @pl.loop(0, NCH)
    def _(j):
        pltpu.sync_copy(ones_v, acc_sh.at[dst_v.at[j]], add=True)

    @pl.when(s < NEXTRA)
    def _():
        pltpu.sync_copy(edst_hbm.at[c, pl.ds(s, 1)], edst_v)
        pltpu.sync_copy(ones_v, acc_sh.at[edst_v.at[0]], add=True)

    plsc.subcore_barrier()

    @pl.when(s < IO_TILES)
    def _():
        sl = pl.ds(s * ROWS_PER_IO, ROWS_PER_IO)
        pltpu.sync_copy(acc_sh.at[sl], out_hbm.at[c, sl])


@functools.partial(
    pl.kernel,
    out_type=jax.ShapeDtypeStruct((NC, N, HIDDEN), jnp.float32),
    mesh=_MESH,
    scratch_types=[
        pltpu.VMEM((SRCBLK, CHUNK), jnp.int32),
        pltpu.VMEM((NCH, CHUNK), jnp.int32),
        pltpu.VMEM((1, CHUNK), jnp.int32),
        pltpu.VMEM((1, CHUNK), jnp.int32),
        pltpu.VMEM((CHUNK, HIDDEN), jnp.float32),
        pltpu.VMEM((CHUNK, HIDDEN), jnp.float32),
        pltpu.VMEM_SHARED((N, HIDDEN), jnp.float32),
        pltpu.SemaphoreType.DMA,
        pltpu.SemaphoreType.DMA,
        pltpu.SemaphoreType.DMA,
        pltpu.SemaphoreType.DMA,
    ],
)
def _sc_aggregate(g_hbm, src_hbm, dst_hbm, esrc_hbm, edst_hbm, zeros_hbm,
                  out_hbm, src_v, dst_v, esrc_v, edst_v, rows0, rows1,
                  acc_sh, sem0, sem1, ssem0, ssem1):
    c = lax.axis_index("c")
    s = lax.axis_index("s")
    pltpu.sync_copy(dst_hbm.at[c, s], dst_v)

    @pl.when(s < IO_TILES)
    def _():
        pltpu.sync_copy(zeros_hbm,
                        acc_sh.at[pl.ds(s * ROWS_PER_IO, ROWS_PER_IO)])

    plsc.subcore_barrier()

    @pl.loop(0, NCH // SRCBLK)
    def _(blk):
        base = blk * SRCBLK
        pltpu.sync_copy(src_hbm.at[c, s, blk], src_v)
        pltpu.make_async_copy(g_hbm.at[src_v.at[0]], rows0, sem0).start()
        pltpu.make_async_copy(g_hbm.at[src_v.at[1]], rows1, sem1).start()

        @pl.loop(0, SRCBLK - 2, step=2)
        def _(j):
            pltpu.make_async_copy(g_hbm.at[src_v.at[j]], rows0, sem0).wait()
            cs0 = pltpu.async_copy(rows0, acc_sh.at[dst_v.at[base + j]],
                                   ssem0, add=True)
            pltpu.make_async_copy(g_hbm.at[src_v.at[j + 1]], rows1, sem1).wait()
            cs1 = pltpu.async_copy(rows1, acc_sh.at[dst_v.at[base + j + 1]],
                                   ssem1, add=True)
            cs0.wait()
            pltpu.make_async_copy(g_hbm.at[src_v.at[j + 2]], rows0, sem0).start()
            cs1.wait()

            @pl.when(j + 3 < SRCBLK)
            def _():
                pltpu.make_async_copy(g_hbm.at[src_v.at[j + 3]], rows1,
                                      sem1).start()

        pltpu.make_async_copy(g_hbm.at[src_v.at[SRCBLK - 2]], rows0, sem0).wait()
        cs0 = pltpu.async_copy(rows0, acc_sh.at[dst_v.at[base + SRCBLK - 2]],
                               ssem0, add=True)
        pltpu.make_async_copy(g_hbm.at[src_v.at[SRCBLK - 1]], rows1, sem1).wait()
        cs1 = pltpu.async_copy(rows1, acc_sh.at[dst_v.at[base + SRCBLK - 1]],
                               ssem1, add=True)
        cs0.wait()
        cs1.wait()

    @pl.when(s < NEXTRA)
    def _():
        pltpu.sync_copy(esrc_hbm.at[c, pl.ds(s, 1)], esrc_v)
        pltpu.sync_copy(edst_hbm.at[c, pl.ds(s, 1)], edst_v)
        pltpu.sync_copy(g_hbm.at[esrc_v.at[0]], rows0)
        pltpu.sync_copy(rows0, acc_sh.at[edst_v.at[0]], add=True)

    plsc.subcore_barrier()

    @pl.when(s < IO_TILES)
    def _():
        sl = pl.ds(s * ROWS_PER_IO, ROWS_PER_IO)
        pltpu.sync_copy(acc_sh.at[sl], out_hbm.at[c, sl])


def _dot(a, b):
    return lax.dot_general(a, b, (((1,), (0,)), ((), ())),
                           preferred_element_type=jnp.float32)


def _dot3(a, b):
    a_hi = a.astype(jnp.bfloat16).astype(jnp.float32)
    a_lo = a - a_hi
    b_hi = b.astype(jnp.bfloat16).astype(jnp.float32)
    b_lo = b - b_hi
    return _dot(a_hi, b_hi) + (_dot(a_hi, b_lo) + _dot(a_lo, b_hi))


R = 2000
NB = N // R

_blk = pl.BlockSpec((R, HIDDEN), lambda i: (i, 0))
_blk_p = pl.BlockSpec((NC, R, HIDDEN), lambda i: (0, i, 0))
_blk_dinv = pl.BlockSpec((R, 1), lambda i: (i, 0))
_rep = lambda shape: pl.BlockSpec(shape, lambda i: tuple(0 for _ in shape))


def _tc_pre_body(h_ref, w_ref, cnt_ref, hw_ref, g_ref, dinv_ref):
    hw = _dot(h_ref[...], w_ref[...])
    deg = 1.0 + cnt_ref[...][0, :, 0] + cnt_ref[...][1, :, 0]
    dinv = lax.rsqrt(deg)[:, None]
    hw_ref[...] = hw
    g_ref[...] = hw * dinv
    dinv_ref[...] = dinv


_tc_pre = pl.pallas_call(
    _tc_pre_body,
    grid=(NB,),
    in_specs=[_blk, _rep((HIDDEN, HIDDEN)), _blk_p],
    out_specs=[_blk, _blk, _blk_dinv],
    out_shape=[
        jax.ShapeDtypeStruct((N, HIDDEN), jnp.float32),
        jax.ShapeDtypeStruct((N, HIDDEN), jnp.float32),
        jax.ShapeDtypeStruct((N, 1), jnp.float32),
    ],
)


def _agg_block(p_ref, hw_ref, dinv_ref, b_ref):
    dinv = dinv_ref[...]
    p = p_ref[...]
    return dinv * (p[0] + p[1]) + hw_ref[...] * (dinv * dinv) + b_ref[...]


def _tc_agg_body(p_ref, hw_ref, dinv_ref, b_ref, agg_ref, st_ref, acc):
    i = pl.program_id(0)

    @pl.when(i == 0)
    def _():
        acc[...] = jnp.zeros((8, HIDDEN), jnp.float32)

    agg = _agg_block(p_ref, hw_ref, dinv_ref, b_ref)
    agg_ref[...] = agg
    s = jnp.sum(agg, axis=0)
    acc[...] += jnp.concatenate(
        [s[None], jnp.zeros((7, HIDDEN), jnp.float32)], axis=0)

    @pl.when(i == NB - 1)
    def _():
        st_ref[...] = acc[...]


_tc_agg = pl.pallas_call(
    _tc_agg_body,
    grid=(NB,),
    in_specs=[_blk_p, _blk, _blk_dinv, _rep((1, HIDDEN))],
    out_specs=[_blk, _rep((8, HIDDEN))],
    out_shape=[
        jax.ShapeDtypeStruct((N, HIDDEN), jnp.float32),
        jax.ShapeDtypeStruct((8, HIDDEN), jnp.float32),
    ],
    scratch_shapes=[pltpu.VMEM((8, HIDDEN), jnp.float32)],
)


def _tc_var_body(agg_ref, st_ref, v_ref, acc):
    i = pl.program_id(0)

    @pl.when(i == 0)
    def _():
        acc[...] = jnp.zeros((8, HIDDEN), jnp.float32)

    m = st_ref[...][0] / N
    d = agg_ref[...] - m
    s2 = jnp.sum(d * d, axis=0)
    acc[...] += jnp.concatenate(
        [s2[None], jnp.zeros((7, HIDDEN), jnp.float32)], axis=0)

    @pl.when(i == NB - 1)
    def _():
        v_ref[...] = acc[...]


_tc_var = pl.pallas_call(
    _tc_var_body,
    grid=(NB,),
    in_specs=[_blk, _rep((8, HIDDEN))],
    out_specs=_rep((8, HIDDEN)),
    out_shape=jax.ShapeDtypeStruct((8, HIDDEN), jnp.float32),
    scratch_shapes=[pltpu.VMEM((8, HIDDEN), jnp.float32)],
)


def _tc_bnmm_body(agg_ref, st_ref, v2_ref, gm_ref, bt_ref, w_ref, dinv_ref,
                  hwn_ref, gn_ref):
    st = st_ref[...]
    m = st[0] / N
    v = v2_ref[...][0] / N
    z = (agg_ref[...] - m) * (lax.rsqrt(v + EPS) * gm_ref[...]) + bt_ref[...]
    h = jnp.maximum(z, 0.0)
    hwn = _dot(h, w_ref[...])
    dinv = dinv_ref[...]
    hwn_ref[...] = hwn
    gn_ref[...] = hwn * dinv


_tc_bnmm = pl.pallas_call(
    _tc_bnmm_body,
    grid=(NB,),
    in_specs=[_blk, _rep((8, HIDDEN)), _rep((8, HIDDEN)), _rep((1, HIDDEN)),
              _rep((1, HIDDEN)), _rep((HIDDEN, HIDDEN)), _blk_dinv],
    out_specs=[_blk, _blk],
    out_shape=[
        jax.ShapeDtypeStruct((N, HIDDEN), jnp.float32),
        jax.ShapeDtypeStruct((N, HIDDEN), jnp.float32),
    ],
)


def _tc_final_body(p_ref, hw_ref, dinv_ref, b_ref, batch_ref,
                   wh1_ref, bh1_ref, gh_ref, bh_ref, wh2_ref, bh2_ref,
                   wo_ref, bo_ref, out_ref, psum, pcnt):
    i = pl.program_id(0)

    @pl.when(i == 0)
    def _():
        psum[...] = jnp.zeros((NUM_GRAPHS, HIDDEN), jnp.float32)
        pcnt[...] = jnp.zeros((NUM_GRAPHS, HIDDEN), jnp.float32)

    agg = _agg_block(p_ref, hw_ref, dinv_ref, b_ref)
    gid = lax.broadcasted_iota(jnp.int32, (NUM_GRAPHS, R), 0)
    batch_row = batch_ref[...].reshape(1, R)
    onehot = (batch_row == gid).astype(jnp.float32)
    psum[...] += _dot3(onehot, agg)
    cnt = jnp.sum(onehot, axis=1, keepdims=True)
    pcnt[...] += jnp.broadcast_to(cnt, (NUM_GRAPHS, HIDDEN))

    @pl.when(i == NB - 1)
    def _():
        pooled = psum[...] / jnp.maximum(pcnt[...][:, :1], 1.0)
        z = _dot(pooled, wh1_ref[...]) + bh1_ref[...]
        m = jnp.mean(z, axis=0)
        v = jnp.mean((z - m) ** 2, axis=0)
        z = (z - m) * lax.rsqrt(v + EPS) * gh_ref[...] + bh_ref[...]
        z = jnp.maximum(z, 0.0)
        z = jnp.maximum(_dot(z, wh2_ref[...]) + bh2_ref[...], 0.0)
        out_ref[...] = _dot(z, wo_ref[...]) + bo_ref[...]


_tc_final = pl.pallas_call(
    _tc_final_body,
    grid=(NB,),
    in_specs=[_blk_p, _blk, _blk_dinv, _rep((1, HIDDEN)),
              pl.BlockSpec((1, 8, R // 8), lambda i: (i, 0, 0)),
              _rep((HIDDEN, HIDDEN)), _rep((1, HIDDEN)), _rep((1, HIDDEN)),
              _rep((1, HIDDEN)), _rep((HIDDEN, HIDDEN)), _rep((1, HIDDEN)),
              _rep((HIDDEN, OUT)), _rep((1, OUT))],
    out_specs=pl.BlockSpec((NUM_GRAPHS, OUT), lambda i: (0, 0)),
    out_shape=jax.ShapeDtypeStruct((NUM_GRAPHS, OUT), jnp.float32),
    scratch_shapes=[pltpu.VMEM((NUM_GRAPHS, HIDDEN), jnp.float32),
                    pltpu.VMEM((NUM_GRAPHS, HIDDEN), jnp.float32)],
)


def kernel(x, laplacian_eigenvector_pe, edge_index, batch, Ws, bs, gammas,
           betas, Wh1, bh1, gh, bh, Wh2, bh2, Wo, bo):
    src2 = edge_index[0].astype(jnp.int32).reshape(NC, 1250, CHUNK)
    dst2 = edge_index[1].astype(jnp.int32).reshape(NC, 1250, CHUNK)
    src = src2[:, :NS * NCH].reshape(NC, NS, NCH // SRCBLK, SRCBLK, CHUNK)
    dst = dst2[:, :NS * NCH].reshape(NC, NS, NCH, CHUNK)
    esrc = src2[:, NS * NCH:]
    edst = dst2[:, NS * NCH:]
    h0 = jnp.concatenate([x, laplacian_eigenvector_pe], axis=1)
    batch2 = batch.astype(jnp.int32).reshape(NB, 8, R // 8)

    ones128 = jnp.ones((CHUNK, HIDDEN), jnp.float32)
    zeros128 = jnp.zeros((ROWS_PER_IO, HIDDEN), jnp.float32)

    counts = _sc_degree(dst, edst, ones128, zeros128)
    hw, g, dinv = _tc_pre(h0, Ws[0], counts)

    for i in range(LAYERS - 1):
        p = _sc_aggregate(g, src, dst, esrc, edst, zeros128)
        agg, st = _tc_agg(p, hw, dinv, bs[i].reshape(1, HIDDEN))
        v2 = _tc_var(agg, st)
        hw, g = _tc_bnmm(agg, st, v2, gammas[i].reshape(1, HIDDEN),
                         betas[i].reshape(1, HIDDEN), Ws[i + 1], dinv)

    p = _sc_aggregate(g, src, dst, esrc, edst, zeros128)
    return _tc_final(p, hw, dinv, bs[LAYERS - 1].reshape(1, HIDDEN), batch2,
                     Wh1, bh1.reshape(1, HIDDEN), gh.reshape(1, HIDDEN),
                     bh.reshape(1, HIDDEN), Wh2, bh2.reshape(1, HIDDEN),
                     Wo, bo.reshape(1, OUT))


# speedup vs baseline: 1.2176x; 1.2176x over previous
import functools

import jax
import jax.numpy as jnp
from jax import lax
from jax.experimental import pallas as pl
from jax.experimental.pallas import tpu as pltpu
from jax.experimental.pallas import tpu_sc as plsc

N = 10000
E = 320000
NUM_GRAPHS = 64
HIDDEN = 128
OUT = 11
PE = 8
IN = 128
LAYERS = 6
EPS = 1e-5

NC = 2
NS = 16
CHUNK = 128
NCH = 78
NEXTRA = 2
SRCBLK = 26
IO_TILES = 10
ROWS_PER_IO = N // IO_TILES

_MESH = plsc.VectorSubcoreMesh(core_axis_name="c", subcore_axis_name="s")
_HIGHEST = lax.Precision.HIGHEST


@functools.partial(
    pl.kernel,
    out_type=jax.ShapeDtypeStruct((NC, N, HIDDEN), jnp.float32),
    mesh=_MESH,
    scratch_types=[
        pltpu.VMEM((NCH, CHUNK), jnp.int32),
        pltpu.VMEM((1, CHUNK), jnp.int32),
        pltpu.VMEM((CHUNK, HIDDEN), jnp.float32),
        pltpu.VMEM_SHARED((N, HIDDEN), jnp.float32),
    ],
)
def _sc_degree(dst_hbm, edst_hbm, ones_hbm, zeros_hbm, out_hbm,
               dst_v, edst_v, ones_v, acc_sh):
    c = lax.axis_index("c")
    s = lax.axis_index("s")
    pltpu.sync_copy(dst_hbm.at[c, s], dst_v)
    pltpu.sync_copy(ones_hbm, ones_v)

    @pl.when(s < IO_TILES)
    def _():
        pltpu.sync_copy(zeros_hbm,
                        acc_sh.at[pl.ds(s * ROWS_PER_IO, ROWS_PER_IO)])

    plsc.subcore_barrier()

    @pl.loop(0, NCH)
    def _(j):
        pltpu.sync_copy(ones_v, acc_sh.at[dst_v.at[j]], add=True)

    @pl.when(s < NEXTRA)
    def _():
        pltpu.sync_copy(edst_hbm.at[c, pl.ds(s, 1)], edst_v)
        pltpu.sync_copy(ones_v, acc_sh.at[edst_v.at[0]], add=True)

    plsc.subcore_barrier()

    @pl.when(s < IO_TILES)
    def _():
        sl = pl.ds(s * ROWS_PER_IO, ROWS_PER_IO)
        pltpu.sync_copy(acc_sh.at[sl], out_hbm.at[c, sl])


@functools.partial(
    pl.kernel,
    out_type=jax.ShapeDtypeStruct((NC, N, HIDDEN), jnp.float32),
    mesh=_MESH,
    scratch_types=[
        pltpu.VMEM((SRCBLK, CHUNK), jnp.int32),
        pltpu.VMEM((NCH, CHUNK), jnp.int32),
        pltpu.VMEM((1, CHUNK), jnp.int32),
        pltpu.VMEM((1, CHUNK), jnp.int32),
        pltpu.VMEM((CHUNK, HIDDEN), jnp.float32),
        pltpu.VMEM((CHUNK, HIDDEN), jnp.float32),
        pltpu.VMEM_SHARED((N, HIDDEN), jnp.float32),
        pltpu.SemaphoreType.DMA,
        pltpu.SemaphoreType.DMA,
    ],
)
def _sc_aggregate(g_hbm, src_hbm, dst_hbm, esrc_hbm, edst_hbm, zeros_hbm,
                  out_hbm, src_v, dst_v, esrc_v, edst_v, rows0, rows1,
                  acc_sh, sem0, sem1):
    c = lax.axis_index("c")
    s = lax.axis_index("s")
    pltpu.sync_copy(dst_hbm.at[c, s], dst_v)

    @pl.when(s < IO_TILES)
    def _():
        pltpu.sync_copy(zeros_hbm,
                        acc_sh.at[pl.ds(s * ROWS_PER_IO, ROWS_PER_IO)])

    plsc.subcore_barrier()

    @pl.loop(0, NCH // SRCBLK)
    def _(blk):
        base = blk * SRCBLK
        pltpu.sync_copy(src_hbm.at[c, s, blk], src_v)
        pltpu.make_async_copy(g_hbm.at[src_v.at[0]], rows0, sem0).start()

        @pl.loop(0, SRCBLK - 2, step=2)
        def _(j):
            pltpu.make_async_copy(g_hbm.at[src_v.at[j + 1]], rows1, sem1).start()
            pltpu.make_async_copy(g_hbm.at[src_v.at[j]], rows0, sem0).wait()
            pltpu.sync_copy(rows0, acc_sh.at[dst_v.at[base + j]], add=True)
            pltpu.make_async_copy(g_hbm.at[src_v.at[j + 2]], rows0, sem0).start()
            pltpu.make_async_copy(g_hbm.at[src_v.at[j + 1]], rows1, sem1).wait()
            pltpu.sync_copy(rows1, acc_sh.at[dst_v.at[base + j + 1]], add=True)

        pltpu.make_async_copy(g_hbm.at[src_v.at[SRCBLK - 1]], rows1, sem1).start()
        pltpu.make_async_copy(g_hbm.at[src_v.at[SRCBLK - 2]], rows0, sem0).wait()
        pltpu.sync_copy(rows0, acc_sh.at[dst_v.at[base + SRCBLK - 2]], add=True)
        pltpu.make_async_copy(g_hbm.at[src_v.at[SRCBLK - 1]], rows1, sem1).wait()
        pltpu.sync_copy(rows1, acc_sh.at[dst_v.at[base + SRCBLK - 1]], add=True)

    @pl.when(s < NEXTRA)
    def _():
        pltpu.sync_copy(esrc_hbm.at[c, pl.ds(s, 1)], esrc_v)
        pltpu.sync_copy(edst_hbm.at[c, pl.ds(s, 1)], edst_v)
        pltpu.sync_copy(g_hbm.at[esrc_v.at[0]], rows0)
        pltpu.sync_copy(rows0, acc_sh.at[edst_v.at[0]], add=True)

    plsc.subcore_barrier()

    @pl.when(s < IO_TILES)
    def _():
        sl = pl.ds(s * ROWS_PER_IO, ROWS_PER_IO)
        pltpu.sync_copy(acc_sh.at[sl], out_hbm.at[c, sl])


def _dot(a, b):
    return lax.dot_general(a, b, (((1,), (0,)), ((), ())),
                           preferred_element_type=jnp.float32)


def _dot3(a, b):
    a_hi = a.astype(jnp.bfloat16).astype(jnp.float32)
    a_lo = a - a_hi
    b_hi = b.astype(jnp.bfloat16).astype(jnp.float32)
    b_lo = b - b_hi
    return _dot(a_hi, b_hi) + (_dot(a_hi, b_lo) + _dot(a_lo, b_hi))


R = 2000
NB = N // R

_blk = pl.BlockSpec((R, HIDDEN), lambda i: (i, 0))
_blk_p = pl.BlockSpec((NC, R, HIDDEN), lambda i: (0, i, 0))
_blk_dinv = pl.BlockSpec((R, 1), lambda i: (i, 0))
_rep = lambda shape: pl.BlockSpec(shape, lambda i: tuple(0 for _ in shape))


def _tc_pre_body(h_ref, w_ref, cnt_ref, hw_ref, g_ref, dinv_ref):
    hw = _dot(h_ref[...], w_ref[...])
    deg = 1.0 + cnt_ref[...][0, :, 0] + cnt_ref[...][1, :, 0]
    dinv = lax.rsqrt(deg)[:, None]
    hw_ref[...] = hw
    g_ref[...] = hw * dinv
    dinv_ref[...] = dinv


_tc_pre = pl.pallas_call(
    _tc_pre_body,
    grid=(NB,),
    in_specs=[_blk, _rep((HIDDEN, HIDDEN)), _blk_p],
    out_specs=[_blk, _blk, _blk_dinv],
    out_shape=[
        jax.ShapeDtypeStruct((N, HIDDEN), jnp.float32),
        jax.ShapeDtypeStruct((N, HIDDEN), jnp.float32),
        jax.ShapeDtypeStruct((N, 1), jnp.float32),
    ],
)


def _agg_block(p_ref, hw_ref, dinv_ref, b_ref):
    dinv = dinv_ref[...]
    p = p_ref[...]
    return dinv * (p[0] + p[1]) + hw_ref[...] * (dinv * dinv) + b_ref[...]


def _tc_agg_body(p_ref, hw_ref, dinv_ref, b_ref, agg_ref, st_ref, acc):
    i = pl.program_id(0)

    @pl.when(i == 0)
    def _():
        acc[...] = jnp.zeros((8, HIDDEN), jnp.float32)

    agg = _agg_block(p_ref, hw_ref, dinv_ref, b_ref)
    agg_ref[...] = agg
    s = jnp.sum(agg, axis=0)
    acc[...] += jnp.concatenate(
        [s[None], jnp.zeros((7, HIDDEN), jnp.float32)], axis=0)

    @pl.when(i == NB - 1)
    def _():
        st_ref[...] = acc[...]


_tc_agg = pl.pallas_call(
    _tc_agg_body,
    grid=(NB,),
    in_specs=[_blk_p, _blk, _blk_dinv, _rep((1, HIDDEN))],
    out_specs=[_blk, _rep((8, HIDDEN))],
    out_shape=[
        jax.ShapeDtypeStruct((N, HIDDEN), jnp.float32),
        jax.ShapeDtypeStruct((8, HIDDEN), jnp.float32),
    ],
    scratch_shapes=[pltpu.VMEM((8, HIDDEN), jnp.float32)],
)


def _tc_var_body(agg_ref, st_ref, v_ref, acc):
    i = pl.program_id(0)

    @pl.when(i == 0)
    def _():
        acc[...] = jnp.zeros((8, HIDDEN), jnp.float32)

    m = st_ref[...][0] / N
    d = agg_ref[...] - m
    s2 = jnp.sum(d * d, axis=0)
    acc[...] += jnp.concatenate(
        [s2[None], jnp.zeros((7, HIDDEN), jnp.float32)], axis=0)

    @pl.when(i == NB - 1)
    def _():
        v_ref[...] = acc[...]


_tc_var = pl.pallas_call(
    _tc_var_body,
    grid=(NB,),
    in_specs=[_blk, _rep((8, HIDDEN))],
    out_specs=_rep((8, HIDDEN)),
    out_shape=jax.ShapeDtypeStruct((8, HIDDEN), jnp.float32),
    scratch_shapes=[pltpu.VMEM((8, HIDDEN), jnp.float32)],
)


def _tc_bnmm_body(agg_ref, st_ref, v2_ref, gm_ref, bt_ref, w_ref, dinv_ref,
                  hwn_ref, gn_ref):
    st = st_ref[...]
    m = st[0] / N
    v = v2_ref[...][0] / N
    z = (agg_ref[...] - m) * (lax.rsqrt(v + EPS) * gm_ref[...]) + bt_ref[...]
    h = jnp.maximum(z, 0.0)
    hwn = _dot(h, w_ref[...])
    dinv = dinv_ref[...]
    hwn_ref[...] = hwn
    gn_ref[...] = hwn * dinv


_tc_bnmm = pl.pallas_call(
    _tc_bnmm_body,
    grid=(NB,),
    in_specs=[_blk, _rep((8, HIDDEN)), _rep((8, HIDDEN)), _rep((1, HIDDEN)),
              _rep((1, HIDDEN)), _rep((HIDDEN, HIDDEN)), _blk_dinv],
    out_specs=[_blk, _blk],
    out_shape=[
        jax.ShapeDtypeStruct((N, HIDDEN), jnp.float32),
        jax.ShapeDtypeStruct((N, HIDDEN), jnp.float32),
    ],
)


def _tc_final_body(p_ref, hw_ref, dinv_ref, b_ref, batch_ref,
                   wh1_ref, bh1_ref, gh_ref, bh_ref, wh2_ref, bh2_ref,
                   wo_ref, bo_ref, out_ref, psum, pcnt):
    i = pl.program_id(0)

    @pl.when(i == 0)
    def _():
        psum[...] = jnp.zeros((NUM_GRAPHS, HIDDEN), jnp.float32)
        pcnt[...] = jnp.zeros((NUM_GRAPHS, HIDDEN), jnp.float32)

    agg = _agg_block(p_ref, hw_ref, dinv_ref, b_ref)
    gid = lax.broadcasted_iota(jnp.int32, (NUM_GRAPHS, R), 0)
    batch_row = batch_ref[...].reshape(1, R)
    onehot = (batch_row == gid).astype(jnp.float32)
    psum[...] += _dot3(onehot, agg)
    cnt = jnp.sum(onehot, axis=1, keepdims=True)
    pcnt[...] += jnp.broadcast_to(cnt, (NUM_GRAPHS, HIDDEN))

    @pl.when(i == NB - 1)
    def _():
        pooled = psum[...] / jnp.maximum(pcnt[...][:, :1], 1.0)
        z = _dot(pooled, wh1_ref[...]) + bh1_ref[...]
        m = jnp.mean(z, axis=0)
        v = jnp.mean((z - m) ** 2, axis=0)
        z = (z - m) * lax.rsqrt(v + EPS) * gh_ref[...] + bh_ref[...]
        z = jnp.maximum(z, 0.0)
        z = jnp.maximum(_dot(z, wh2_ref[...]) + bh2_ref[...], 0.0)
        out_ref[...] = _dot(z, wo_ref[...]) + bo_ref[...]


_tc_final = pl.pallas_call(
    _tc_final_body,
    grid=(NB,),
    in_specs=[_blk_p, _blk, _blk_dinv, _rep((1, HIDDEN)),
              pl.BlockSpec((1, 8, R // 8), lambda i: (i, 0, 0)),
              _rep((HIDDEN, HIDDEN)), _rep((1, HIDDEN)), _rep((1, HIDDEN)),
              _rep((1, HIDDEN)), _rep((HIDDEN, HIDDEN)), _rep((1, HIDDEN)),
              _rep((HIDDEN, OUT)), _rep((1, OUT))],
    out_specs=pl.BlockSpec((NUM_GRAPHS, OUT), lambda i: (0, 0)),
    out_shape=jax.ShapeDtypeStruct((NUM_GRAPHS, OUT), jnp.float32),
    scratch_shapes=[pltpu.VMEM((NUM_GRAPHS, HIDDEN), jnp.float32),
                    pltpu.VMEM((NUM_GRAPHS, HIDDEN), jnp.float32)],
)


def kernel(x, laplacian_eigenvector_pe, edge_index, batch, Ws, bs, gammas,
           betas, Wh1, bh1, gh, bh, Wh2, bh2, Wo, bo):
    src2 = edge_index[0].astype(jnp.int32).reshape(NC, 1250, CHUNK)
    dst2 = edge_index[1].astype(jnp.int32).reshape(NC, 1250, CHUNK)
    src = src2[:, :NS * NCH].reshape(NC, NS, NCH // SRCBLK, SRCBLK, CHUNK)
    dst = dst2[:, :NS * NCH].reshape(NC, NS, NCH, CHUNK)
    esrc = src2[:, NS * NCH:]
    edst = dst2[:, NS * NCH:]
    h0 = jnp.concatenate([x, laplacian_eigenvector_pe], axis=1)
    batch2 = batch.astype(jnp.int32).reshape(NB, 8, R // 8)

    ones128 = jnp.ones((CHUNK, HIDDEN), jnp.float32)
    zeros128 = jnp.zeros((ROWS_PER_IO, HIDDEN), jnp.float32)

    counts = _sc_degree(dst, edst, ones128, zeros128)
    hw, g, dinv = _tc_pre(h0, Ws[0], counts)

    for i in range(LAYERS - 1):
        p = _sc_aggregate(g, src, dst, esrc, edst, zeros128)
        agg, st = _tc_agg(p, hw, dinv, bs[i].reshape(1, HIDDEN))
        v2 = _tc_var(agg, st)
        hw, g = _tc_bnmm(agg, st, v2, gammas[i].reshape(1, HIDDEN),
                         betas[i].reshape(1, HIDDEN), Ws[i + 1], dinv)

    p = _sc_aggregate(g, src, dst, esrc, edst, zeros128)
    return _tc_final(p, hw, dinv, bs[LAYERS - 1].reshape(1, HIDDEN), batch2,
                     Wh1, bh1.reshape(1, HIDDEN), gh.reshape(1, HIDDEN),
                     bh.reshape(1, HIDDEN), Wh2, bh2.reshape(1, HIDDEN),
                     Wo, bo.reshape(1, OUT))


# speedup vs baseline: 1.2277x; 1.0083x over previous
import functools

import jax
import jax.numpy as jnp
from jax import lax
from jax.experimental import pallas as pl
from jax.experimental.pallas import tpu as pltpu
from jax.experimental.pallas import tpu_sc as plsc

N = 10000
E = 320000
NUM_GRAPHS = 64
HIDDEN = 128
OUT = 11
PE = 8
IN = 128
LAYERS = 6
EPS = 1e-5

NC = 2
NS = 16
CHUNK = 128
NCH = 78
NEXTRA = 2
SRCBLK = 26
IO_TILES = 10
ROWS_PER_IO = N // IO_TILES

_MESH = plsc.VectorSubcoreMesh(core_axis_name="c", subcore_axis_name="s")
_HIGHEST = lax.Precision.HIGHEST


@functools.partial(
    pl.kernel,
    out_type=jax.ShapeDtypeStruct((NC, N, HIDDEN), jnp.float32),
    mesh=_MESH,
    scratch_types=[
        pltpu.VMEM((NCH, CHUNK), jnp.int32),
        pltpu.VMEM((1, CHUNK), jnp.int32),
        pltpu.VMEM((CHUNK, HIDDEN), jnp.float32),
        pltpu.VMEM_SHARED((N, HIDDEN), jnp.float32),
    ],
)
def _sc_degree(dst_hbm, edst_hbm, ones_hbm, zeros_hbm, out_hbm,
               dst_v, edst_v, ones_v, acc_sh):
    c = lax.axis_index("c")
    s = lax.axis_index("s")
    pltpu.sync_copy(dst_hbm.at[c, s], dst_v)
    pltpu.sync_copy(ones_hbm, ones_v)

    @pl.when(s < IO_TILES)
    def _():
        pltpu.sync_copy(zeros_hbm,
                        acc_sh.at[pl.ds(s * ROWS_PER_IO, ROWS_PER_IO)])

    plsc.subcore_barrier()

    @pl.loop(0, NCH)
    def _(j):
        pltpu.sync_copy(ones_v, acc_sh.at[dst_v.at[j]], add=True)

    @pl.when(s < NEXTRA)
    def _():
        pltpu.sync_copy(edst_hbm.at[c, pl.ds(s, 1)], edst_v)
        pltpu.sync_copy(ones_v, acc_sh.at[edst_v.at[0]], add=True)

    plsc.subcore_barrier()

    @pl.when(s < IO_TILES)
    def _():
        sl = pl.ds(s * ROWS_PER_IO, ROWS_PER_IO)
        pltpu.sync_copy(acc_sh.at[sl], out_hbm.at[c, sl])


@functools.partial(
    pl.kernel,
    out_type=jax.ShapeDtypeStruct((NC, N, HIDDEN), jnp.float32),
    mesh=_MESH,
    scratch_types=[
        pltpu.VMEM((SRCBLK, CHUNK), jnp.int32),
        pltpu.VMEM((NCH, CHUNK), jnp.int32),
        pltpu.VMEM((1, CHUNK), jnp.int32),
        pltpu.VMEM((1, CHUNK), jnp.int32),
        pltpu.VMEM((CHUNK, HIDDEN), jnp.float32),
        pltpu.VMEM((CHUNK, HIDDEN), jnp.float32),
        pltpu.VMEM_SHARED((N, HIDDEN), jnp.float32),
        pltpu.SemaphoreType.DMA,
        pltpu.SemaphoreType.DMA,
    ],
)
def _sc_aggregate(g_hbm, src_hbm, dst_hbm, esrc_hbm, edst_hbm, zeros_hbm,
                  out_hbm, src_v, dst_v, esrc_v, edst_v, rows0, rows1,
                  acc_sh, sem0, sem1):
    c = lax.axis_index("c")
    s = lax.axis_index("s")
    pltpu.sync_copy(dst_hbm.at[c, s], dst_v)

    @pl.when(s < IO_TILES)
    def _():
        pltpu.sync_copy(zeros_hbm,
                        acc_sh.at[pl.ds(s * ROWS_PER_IO, ROWS_PER_IO)])

    plsc.subcore_barrier()

    @pl.loop(0, NCH // SRCBLK)
    def _(blk):
        base = blk * SRCBLK
        pltpu.sync_copy(src_hbm.at[c, s, blk], src_v)
        pltpu.make_async_copy(g_hbm.at[src_v.at[0]], rows0, sem0).start()

        @pl.loop(0, SRCBLK - 2, step=2)
        def _(j):
            pltpu.make_async_copy(g_hbm.at[src_v.at[j + 1]], rows1, sem1).start()
            pltpu.make_async_copy(g_hbm.at[src_v.at[j]], rows0, sem0).wait()
            pltpu.sync_copy(rows0, acc_sh.at[dst_v.at[base + j]], add=True)
            pltpu.make_async_copy(g_hbm.at[src_v.at[j + 2]], rows0, sem0).start()
            pltpu.make_async_copy(g_hbm.at[src_v.at[j + 1]], rows1, sem1).wait()
            pltpu.sync_copy(rows1, acc_sh.at[dst_v.at[base + j + 1]], add=True)

        pltpu.make_async_copy(g_hbm.at[src_v.at[SRCBLK - 1]], rows1, sem1).start()
        pltpu.make_async_copy(g_hbm.at[src_v.at[SRCBLK - 2]], rows0, sem0).wait()
        pltpu.sync_copy(rows0, acc_sh.at[dst_v.at[base + SRCBLK - 2]], add=True)
        pltpu.make_async_copy(g_hbm.at[src_v.at[SRCBLK - 1]], rows1, sem1).wait()
        pltpu.sync_copy(rows1, acc_sh.at[dst_v.at[base + SRCBLK - 1]], add=True)

    @pl.when(s < NEXTRA)
    def _():
        pltpu.sync_copy(esrc_hbm.at[c, pl.ds(s, 1)], esrc_v)
        pltpu.sync_copy(edst_hbm.at[c, pl.ds(s, 1)], edst_v)
        pltpu.sync_copy(g_hbm.at[esrc_v.at[0]], rows0)
        pltpu.sync_copy(rows0, acc_sh.at[edst_v.at[0]], add=True)

    plsc.subcore_barrier()

    @pl.when(s < IO_TILES)
    def _():
        sl = pl.ds(s * ROWS_PER_IO, ROWS_PER_IO)
        pltpu.sync_copy(acc_sh.at[sl], out_hbm.at[c, sl])


def _dot(a, b):
    return lax.dot_general(a, b, (((1,), (0,)), ((), ())),
                           preferred_element_type=jnp.float32)


def _dot3(a, b):
    a_hi = a.astype(jnp.bfloat16).astype(jnp.float32)
    a_lo = a - a_hi
    b_hi = b.astype(jnp.bfloat16).astype(jnp.float32)
    b_lo = b - b_hi
    return _dot(a_hi, b_hi) + (_dot(a_hi, b_lo) + _dot(a_lo, b_hi))


R = 2000
NB = N // R

_blk = pl.BlockSpec((R, HIDDEN), lambda i: (i, 0))
_blk_p = pl.BlockSpec((NC, R, HIDDEN), lambda i: (0, i, 0))
_blk_dinv = pl.BlockSpec((R, 1), lambda i: (i, 0))
_rep = lambda shape: pl.BlockSpec(shape, lambda i: tuple(0 for _ in shape))


def _tc_pre_body(h_ref, w_ref, cnt_ref, hw_ref, g_ref, dinv_ref):
    hw = _dot(h_ref[...], w_ref[...])
    deg = 1.0 + cnt_ref[...][0, :, 0] + cnt_ref[...][1, :, 0]
    dinv = lax.rsqrt(deg)[:, None]
    hw_ref[...] = hw
    g_ref[...] = hw * dinv
    dinv_ref[...] = dinv


_tc_pre = pl.pallas_call(
    _tc_pre_body,
    grid=(NB,),
    in_specs=[_blk, _rep((HIDDEN, HIDDEN)), _blk_p],
    out_specs=[_blk, _blk, _blk_dinv],
    out_shape=[
        jax.ShapeDtypeStruct((N, HIDDEN), jnp.float32),
        jax.ShapeDtypeStruct((N, HIDDEN), jnp.float32),
        jax.ShapeDtypeStruct((N, 1), jnp.float32),
    ],
)


def _agg_block(p_ref, hw_ref, dinv_ref, b_ref):
    dinv = dinv_ref[...]
    p = p_ref[...]
    return dinv * (p[0] + p[1]) + hw_ref[...] * (dinv * dinv) + b_ref[...]


def _tc_layer_body(p_ref, hw_ref, dinv_ref, b_ref, gm_ref, bt_ref, w_ref,
                   hwn_ref, gn_ref, agg_vm, st_vm):
    i = pl.program_id(0)
    b = lax.rem(i, NB)

    @pl.when(i == 0)
    def _():
        st_vm[...] = jnp.zeros((8, HIDDEN), jnp.float32)

    @pl.when(i < NB)
    def _():
        agg = _agg_block(p_ref, hw_ref, dinv_ref, b_ref)
        agg_vm[pl.ds(b * R, R), :] = agg
        s = jnp.sum(agg, axis=0)
        st_vm[...] += jnp.concatenate(
            [s[None], jnp.zeros((7, HIDDEN), jnp.float32)], axis=0)

    @pl.when((i >= NB) & (i < 2 * NB))
    def _():
        m = st_vm[...][0] / N
        d = agg_vm[pl.ds(b * R, R), :] - m
        s2 = jnp.sum(d * d, axis=0)
        st_vm[...] += jnp.concatenate(
            [jnp.zeros((1, HIDDEN), jnp.float32), s2[None],
             jnp.zeros((6, HIDDEN), jnp.float32)], axis=0)

    @pl.when(i >= 2 * NB)
    def _():
        st = st_vm[...]
        m = st[0] / N
        v = st[1] / N
        z = (agg_vm[pl.ds(b * R, R), :] - m) * (lax.rsqrt(v + EPS)
                                                * gm_ref[...]) + bt_ref[...]
        h = jnp.maximum(z, 0.0)
        hwn = _dot(h, w_ref[...])
        hwn_ref[...] = hwn
        gn_ref[...] = hwn * dinv_ref[...]


def _clamp_blk(shape):
    return pl.BlockSpec(shape, lambda i: (jnp.minimum(i, NB - 1), 0))


_tc_layer = pl.pallas_call(
    _tc_layer_body,
    grid=(3 * NB,),
    in_specs=[pl.BlockSpec((NC, R, HIDDEN), lambda i: (0, jnp.minimum(i, NB - 1), 0)),
              _clamp_blk((R, HIDDEN)),
              pl.BlockSpec((R, 1), lambda i: (lax.rem(i, NB), 0)),
              _rep((1, HIDDEN)), _rep((1, HIDDEN)), _rep((1, HIDDEN)),
              _rep((HIDDEN, HIDDEN))],
    out_specs=[pl.BlockSpec((R, HIDDEN), lambda i: (lax.rem(i, NB), 0)),
               pl.BlockSpec((R, HIDDEN), lambda i: (lax.rem(i, NB), 0))],
    out_shape=[
        jax.ShapeDtypeStruct((N, HIDDEN), jnp.float32),
        jax.ShapeDtypeStruct((N, HIDDEN), jnp.float32),
    ],
    scratch_shapes=[pltpu.VMEM((N, HIDDEN), jnp.float32),
                    pltpu.VMEM((8, HIDDEN), jnp.float32)],
)


def _tc_agg_body(p_ref, hw_ref, dinv_ref, b_ref, agg_ref, st_ref, acc):
    i = pl.program_id(0)

    @pl.when(i == 0)
    def _():
        acc[...] = jnp.zeros((8, HIDDEN), jnp.float32)

    agg = _agg_block(p_ref, hw_ref, dinv_ref, b_ref)
    agg_ref[...] = agg
    s = jnp.sum(agg, axis=0)
    acc[...] += jnp.concatenate(
        [s[None], jnp.zeros((7, HIDDEN), jnp.float32)], axis=0)

    @pl.when(i == NB - 1)
    def _():
        st_ref[...] = acc[...]


_tc_agg = pl.pallas_call(
    _tc_agg_body,
    grid=(NB,),
    in_specs=[_blk_p, _blk, _blk_dinv, _rep((1, HIDDEN))],
    out_specs=[_blk, _rep((8, HIDDEN))],
    out_shape=[
        jax.ShapeDtypeStruct((N, HIDDEN), jnp.float32),
        jax.ShapeDtypeStruct((8, HIDDEN), jnp.float32),
    ],
    scratch_shapes=[pltpu.VMEM((8, HIDDEN), jnp.float32)],
)


def _tc_var_body(agg_ref, st_ref, v_ref, acc):
    i = pl.program_id(0)

    @pl.when(i == 0)
    def _():
        acc[...] = jnp.zeros((8, HIDDEN), jnp.float32)

    m = st_ref[...][0] / N
    d = agg_ref[...] - m
    s2 = jnp.sum(d * d, axis=0)
    acc[...] += jnp.concatenate(
        [s2[None], jnp.zeros((7, HIDDEN), jnp.float32)], axis=0)

    @pl.when(i == NB - 1)
    def _():
        v_ref[...] = acc[...]


_tc_var = pl.pallas_call(
    _tc_var_body,
    grid=(NB,),
    in_specs=[_blk, _rep((8, HIDDEN))],
    out_specs=_rep((8, HIDDEN)),
    out_shape=jax.ShapeDtypeStruct((8, HIDDEN), jnp.float32),
    scratch_shapes=[pltpu.VMEM((8, HIDDEN), jnp.float32)],
)


def _tc_bnmm_body(agg_ref, st_ref, v2_ref, gm_ref, bt_ref, w_ref, dinv_ref,
                  hwn_ref, gn_ref):
    st = st_ref[...]
    m = st[0] / N
    v = v2_ref[...][0] / N
    z = (agg_ref[...] - m) * (lax.rsqrt(v + EPS) * gm_ref[...]) + bt_ref[...]
    h = jnp.maximum(z, 0.0)
    hwn = _dot(h, w_ref[...])
    dinv = dinv_ref[...]
    hwn_ref[...] = hwn
    gn_ref[...] = hwn * dinv


_tc_bnmm = pl.pallas_call(
    _tc_bnmm_body,
    grid=(NB,),
    in_specs=[_blk, _rep((8, HIDDEN)), _rep((8, HIDDEN)), _rep((1, HIDDEN)),
              _rep((1, HIDDEN)), _rep((HIDDEN, HIDDEN)), _blk_dinv],
    out_specs=[_blk, _blk],
    out_shape=[
        jax.ShapeDtypeStruct((N, HIDDEN), jnp.float32),
        jax.ShapeDtypeStruct((N, HIDDEN), jnp.float32),
    ],
)


def _tc_final_body(p_ref, hw_ref, dinv_ref, b_ref, batch_ref,
                   wh1_ref, bh1_ref, gh_ref, bh_ref, wh2_ref, bh2_ref,
                   wo_ref, bo_ref, out_ref, psum, pcnt):
    i = pl.program_id(0)

    @pl.when(i == 0)
    def _():
        psum[...] = jnp.zeros((NUM_GRAPHS, HIDDEN), jnp.float32)
        pcnt[...] = jnp.zeros((NUM_GRAPHS, HIDDEN), jnp.float32)

    agg = _agg_block(p_ref, hw_ref, dinv_ref, b_ref)
    gid = lax.broadcasted_iota(jnp.int32, (NUM_GRAPHS, R), 0)
    batch_row = batch_ref[...].reshape(1, R)
    onehot = (batch_row == gid).astype(jnp.float32)
    psum[...] += _dot3(onehot, agg)
    cnt = jnp.sum(onehot, axis=1, keepdims=True)
    pcnt[...] += jnp.broadcast_to(cnt, (NUM_GRAPHS, HIDDEN))

    @pl.when(i == NB - 1)
    def _():
        pooled = psum[...] / jnp.maximum(pcnt[...][:, :1], 1.0)
        z = _dot(pooled, wh1_ref[...]) + bh1_ref[...]
        m = jnp.mean(z, axis=0)
        v = jnp.mean((z - m) ** 2, axis=0)
        z = (z - m) * lax.rsqrt(v + EPS) * gh_ref[...] + bh_ref[...]
        z = jnp.maximum(z, 0.0)
        z = jnp.maximum(_dot(z, wh2_ref[...]) + bh2_ref[...], 0.0)
        out_ref[...] = _dot(z, wo_ref[...]) + bo_ref[...]


_tc_final = pl.pallas_call(
    _tc_final_body,
    grid=(NB,),
    in_specs=[_blk_p, _blk, _blk_dinv, _rep((1, HIDDEN)),
              pl.BlockSpec((1, 8, R // 8), lambda i: (i, 0, 0)),
              _rep((HIDDEN, HIDDEN)), _rep((1, HIDDEN)), _rep((1, HIDDEN)),
              _rep((1, HIDDEN)), _rep((HIDDEN, HIDDEN)), _rep((1, HIDDEN)),
              _rep((HIDDEN, OUT)), _rep((1, OUT))],
    out_specs=pl.BlockSpec((NUM_GRAPHS, OUT), lambda i: (0, 0)),
    out_shape=jax.ShapeDtypeStruct((NUM_GRAPHS, OUT), jnp.float32),
    scratch_shapes=[pltpu.VMEM((NUM_GRAPHS, HIDDEN), jnp.float32),
                    pltpu.VMEM((NUM_GRAPHS, HIDDEN), jnp.float32)],
)


def kernel(x, laplacian_eigenvector_pe, edge_index, batch, Ws, bs, gammas,
           betas, Wh1, bh1, gh, bh, Wh2, bh2, Wo, bo):
    src2 = edge_index[0].astype(jnp.int32).reshape(NC, 1250, CHUNK)
    dst2 = edge_index[1].astype(jnp.int32).reshape(NC, 1250, CHUNK)
    src = src2[:, :NS * NCH].reshape(NC, NS, NCH // SRCBLK, SRCBLK, CHUNK)
    dst = dst2[:, :NS * NCH].reshape(NC, NS, NCH, CHUNK)
    esrc = src2[:, NS * NCH:]
    edst = dst2[:, NS * NCH:]
    h0 = jnp.concatenate([x, laplacian_eigenvector_pe], axis=1)
    batch2 = batch.astype(jnp.int32).reshape(NB, 8, R // 8)

    ones128 = jnp.ones((CHUNK, HIDDEN), jnp.float32)
    zeros128 = jnp.zeros((ROWS_PER_IO, HIDDEN), jnp.float32)

    counts = _sc_degree(dst, edst, ones128, zeros128)
    hw, g, dinv = _tc_pre(h0, Ws[0], counts)

    for i in range(LAYERS - 1):
        p = _sc_aggregate(g, src, dst, esrc, edst, zeros128)
        hw, g = _tc_layer(p, hw, dinv, bs[i].reshape(1, HIDDEN),
                          gammas[i].reshape(1, HIDDEN),
                          betas[i].reshape(1, HIDDEN), Ws[i + 1])

    p = _sc_aggregate(g, src, dst, esrc, edst, zeros128)
    return _tc_final(p, hw, dinv, bs[LAYERS - 1].reshape(1, HIDDEN), batch2,
                     Wh1, bh1.reshape(1, HIDDEN), gh.reshape(1, HIDDEN),
                     bh.reshape(1, HIDDEN), Wh2, bh2.reshape(1, HIDDEN),
                     Wo, bo.reshape(1, OUT))
